# baseline (speedup 1.0000x reference)
import functools

import jax
import jax.numpy as jnp
from jax import lax
from jax.experimental import pallas as pl
from jax.experimental.pallas import tpu as pltpu
from jax.experimental.pallas import tpu_sc as plsc

B = 4096
L = 200
D = 128
P1 = 64
P2 = 128
LANES = 16
NCHUNK = D // LANES
NBUF = 3

_info = plsc.get_sparse_core_info()
_NC = _info.num_cores
_NS = _info.num_subcores
NW = _NC * _NS
RPW = B // NW


@functools.partial(
    pl.kernel,
    out_type=jax.ShapeDtypeStruct((B, D), jnp.float32),
    mesh=plsc.VectorSubcoreMesh(core_axis_name="c", subcore_axis_name="s"),
    scratch_types=[
        pltpu.VMEM((RPW, L), jnp.int32),
        pltpu.VMEM((NBUF, L, D), jnp.float32),
        pltpu.VMEM((RPW, D), jnp.float32),
        pltpu.SemaphoreType.DMA,
        pltpu.SemaphoreType.DMA,
        pltpu.SemaphoreType.DMA,
        pltpu.SemaphoreType.DMA,
        pltpu.SemaphoreType.DMA,
        pltpu.SemaphoreType.DMA,
        pltpu.SemaphoreType.DMA,
        pltpu.SemaphoreType.DMA,
        pltpu.SemaphoreType.DMA,
    ],
)
def _bow_max_kernel(idx_hbm, table_hbm, out_hbm,
                    idx_v, rows_v, out_v,
                    sem0a, sem1a, sem2a, sem0b, sem1b, sem2b,
                    sem0c, sem1c, sem2c):
    wid = lax.axis_index("s") * _NC + lax.axis_index("c")
    base = wid * RPW

    pltpu.sync_copy(idx_hbm.at[pl.ds(base, RPW), :], idx_v)

    sems_a = (sem0a, sem1a, sem2a)
    sems_b = (sem0b, sem1b, sem2b)
    sems_c = (sem0c, sem1c, sem2c)

    def gather(r, slot):
        a = pltpu.make_async_copy(
            table_hbm.at[idx_v.at[r, pl.ds(0, P1)]],
            rows_v.at[slot, pl.ds(0, P1)], sems_a[slot])
        b = pltpu.make_async_copy(
            table_hbm.at[idx_v.at[r, pl.ds(P1, P2 - P1)]],
            rows_v.at[slot, pl.ds(P1, P2 - P1)], sems_b[slot])
        c = pltpu.make_async_copy(
            table_hbm.at[idx_v.at[r, pl.ds(P2, L - P2)]],
            rows_v.at[slot, pl.ds(P2, L - P2)], sems_c[slot])
        return a, b, c

    def start_gather(r, slot):
        for d in gather(r, slot):
            d.start()

    for p in range(NBUF - 1):
        start_gather(p, p)

    def do_row(r, slot):
        @pl.when(r < RPW - (NBUF - 1))
        def _():
            start_gather(r + NBUF - 1, (slot + NBUF - 1) % NBUF)

        wa, wb, wc = gather(r, slot)

        def body(j, accs):
            return tuple(
                jnp.maximum(accs[c], rows_v[slot, j, pl.ds(c * LANES, LANES)])
                for c in range(NCHUNK))

        wa.wait()
        init = tuple(
            rows_v[slot, 0, pl.ds(c * LANES, LANES)] for c in range(NCHUNK))
        accs = lax.fori_loop(1, P1, body, init)
        wb.wait()
        accs = lax.fori_loop(P1, P2, body, accs)
        wc.wait()
        accs = lax.fori_loop(P2, L, body, accs)
        for c in range(NCHUNK):
            out_v[r, pl.ds(c * LANES, LANES)] = accs[c]

    def outer(g, _):
        for b in range(NBUF):
            do_row(NBUF * g + b, b)
        return 0

    n_full = RPW // NBUF
    lax.fori_loop(0, n_full, outer, 0)
    for b in range(RPW - n_full * NBUF):
        do_row(n_full * NBUF + b, b)

    pltpu.sync_copy(out_v, out_hbm.at[pl.ds(base, RPW), :])


def kernel(inputs, emb_weight):
    return _bow_max_kernel(inputs, emb_weight)
